# Initial kernel scaffold; baseline (speedup 1.0000x reference)
import jax
import jax.numpy as jnp
from jax.experimental import pallas as pl


def kernel(adj_his, t_diff, xu_t_plus, xi_t_plus, xu_embed, xi_embed, W):
    raise NotImplementedError("write your pallas kernel here")


# speedup vs baseline: 7.6940x; 7.6940x over previous
import functools

import jax
import jax.numpy as jnp
from jax import lax
from jax.experimental import pallas as pl
from jax.experimental.pallas import tpu as pltpu, tpu_sc as plsc

_N_USER = 6000
_N_ITEM = 4000
_N = _N_USER + _N_ITEM
_D = 128
_E = 320000

_NC = 2
_NS = 16
_NW = _NC * _NS
_EPW = _E // _NW
_C = 80
_CHUNKS = _EPW // _C
_NPAD = 10240
_RPT = _NPAD // _NS


def _pre_body(xt_ref, xe_ref, w_ref, y_ref, norm_ref):
    xt = xt_ref[...]
    norm = jnp.sqrt(jnp.max(jnp.sum(xt * xt, axis=1)))
    inv = 1.0 / norm
    y_ref[...] = jnp.dot(xe_ref[...], w_ref[...],
                         preferred_element_type=jnp.float32) * inv
    norm_ref[...] = jnp.full((1, 1), norm, jnp.float32)


_pre = pl.pallas_call(
    _pre_body,
    out_shape=(
        jax.ShapeDtypeStruct((_N, _D), jnp.float32),
        jax.ShapeDtypeStruct((1, 1), jnp.float32),
    ),
)


_mesh = plsc.VectorSubcoreMesh(
    core_axis_name="c", subcore_axis_name="s", num_cores=_NC, num_subcores=_NS
)


@functools.partial(
    pl.kernel,
    out_type=(
        jax.ShapeDtypeStruct((_NC, _NPAD, _D), jnp.float32),
        jax.ShapeDtypeStruct((_NC, _NPAD), jnp.float32),
    ),
    mesh=_mesh,
    scratch_types=[
        pltpu.VMEM((_CHUNKS, _C), jnp.int32),
        pltpu.VMEM((_CHUNKS, _C), jnp.int32),
        pltpu.VMEM((_C, _D), jnp.float32),
        pltpu.VMEM((_C,), jnp.float32),
        pltpu.VMEM_SHARED((_NPAD, _D), jnp.float32),
        pltpu.VMEM_SHARED((_NPAD,), jnp.float32),
        pltpu.SemaphoreType.DMA,
    ],
)
def _sc_agg(y_hbm, src_hbm, dst_hbm, zrows_hbm, zdeg_hbm,
            acc_out, deg_out,
            src_v, dst_v, rows_v, ones_v, acc_sh, deg_sh, gsem):
    c = lax.axis_index("c")
    s = lax.axis_index("s")
    w = c * _NS + s

    pltpu.sync_copy(src_hbm.at[w], src_v)
    pltpu.sync_copy(dst_hbm.at[w], dst_v)
    for i in range(_C // 16):
        ones_v[pl.ds(i * 16, 16)] = jnp.ones((16,), jnp.float32)

    pltpu.sync_copy(zrows_hbm, acc_sh.at[pl.ds(s * _RPT, _RPT)])
    pltpu.sync_copy(zdeg_hbm, deg_sh.at[pl.ds(s * _RPT, _RPT)])
    plsc.subcore_barrier()

    def body(k, carry):
        pltpu.async_copy(y_hbm.at[src_v.at[k]], rows_v, gsem).wait()
        pltpu.sync_copy(rows_v, acc_sh.at[dst_v.at[k]], add=True)
        pltpu.sync_copy(ones_v, deg_sh.at[dst_v.at[k]], add=True)
        return carry

    lax.fori_loop(0, _CHUNKS, body, 0)

    plsc.subcore_barrier()
    sl = pl.ds(s * _RPT, _RPT)
    pltpu.sync_copy(acc_sh.at[sl], acc_out.at[c, sl])
    pltpu.sync_copy(deg_sh.at[sl], deg_out.at[c, sl])


def _post_body(xt_ref, acc_ref, deg_ref, norm_ref, td_ref, out_ref):
    inv = 1.0 / norm_ref[0, 0]
    agg = (acc_ref[0] + acc_ref[1])[:_N]
    deg = (deg_ref[0] + deg_ref[1])[:_N]
    scale = lax.rsqrt(deg + 1.0)
    out_ref[...] = xt_ref[...] * inv + td_ref[0, 0] * jnp.tanh(agg * scale)


_post = pl.pallas_call(
    _post_body,
    in_specs=[
        pl.BlockSpec(memory_space=pltpu.VMEM),
        pl.BlockSpec(memory_space=pltpu.VMEM),
        pl.BlockSpec(memory_space=pltpu.VMEM),
        pl.BlockSpec(memory_space=pltpu.SMEM),
        pl.BlockSpec(memory_space=pltpu.SMEM),
    ],
    out_shape=jax.ShapeDtypeStruct((_N, _D), jnp.float32),
)


def kernel(adj_his, t_diff, xu_t_plus, xi_t_plus, xu_embed, xi_embed, W):
    xt = jnp.concatenate([xu_t_plus, xi_t_plus], axis=0)
    xe = jnp.concatenate([xu_embed, xi_embed], axis=0)
    adj = adj_his.astype(jnp.int32)
    src = adj[0].reshape(_NW, _CHUNKS, _C)
    dst = adj[1].reshape(_NW, _CHUNKS, _C)

    y, norm = _pre(xt, xe, W)
    zrows = jnp.zeros((_RPT, _D), jnp.float32)
    zdeg = jnp.zeros((_RPT,), jnp.float32)
    acc, deg = _sc_agg(y, src, dst, zrows, zdeg)

    out = _post(xt, acc, deg.reshape(_NC, _NPAD, 1),
                norm, t_diff.reshape(1, 1))
    return out[:_N_USER], out[_N_USER:]


# speedup vs baseline: 9.3896x; 1.2204x over previous
import functools

import jax
import jax.numpy as jnp
from jax import lax
from jax.experimental import pallas as pl
from jax.experimental.pallas import tpu as pltpu, tpu_sc as plsc

_N_USER = 6000
_N_ITEM = 4000
_N = _N_USER + _N_ITEM
_D = 128
_E = 320000

_NC = 2
_NS = 16
_NW = _NC * _NS
_EPW = _E // _NW
_C = 80
_CHUNKS = _EPW // _C
_NPAD = 10240
_RPT = _NPAD // _NS


def _pre_body(xt_ref, xe_ref, w_ref, y_ref, norm_ref):
    xt = xt_ref[...]
    norm = jnp.sqrt(jnp.max(jnp.sum(xt * xt, axis=1)))
    inv = 1.0 / norm
    y_ref[...] = jnp.dot(xe_ref[...], w_ref[...],
                         preferred_element_type=jnp.float32) * inv
    norm_ref[...] = jnp.full((1, 1), norm, jnp.float32)


_pre = pl.pallas_call(
    _pre_body,
    out_shape=(
        jax.ShapeDtypeStruct((_N, _D), jnp.float32),
        jax.ShapeDtypeStruct((1, 1), jnp.float32),
    ),
)


_mesh = plsc.VectorSubcoreMesh(
    core_axis_name="c", subcore_axis_name="s", num_cores=_NC, num_subcores=_NS
)


@functools.partial(
    pl.kernel,
    out_type=(
        jax.ShapeDtypeStruct((_NC * _NPAD, _D), jnp.float32),
        jax.ShapeDtypeStruct((_NC * _NPAD,), jnp.float32),
    ),
    mesh=_mesh,
    scratch_types=[
        pltpu.VMEM((_CHUNKS, _C), jnp.int32),
        pltpu.VMEM((_C,), jnp.int32),
        pltpu.VMEM((_C,), jnp.int32),
        pltpu.VMEM((_C, _D), jnp.float32),
        pltpu.VMEM((_C, _D), jnp.float32),
        pltpu.VMEM((_C,), jnp.float32),
        pltpu.VMEM_SHARED((_NPAD, _D), jnp.float32),
        pltpu.VMEM_SHARED((_NPAD,), jnp.float32),
        pltpu.SemaphoreType.DMA,
        pltpu.SemaphoreType.DMA,
        pltpu.SemaphoreType.DMA,
        pltpu.SemaphoreType.DMA,
        pltpu.SemaphoreType.DMA,
    ],
)
def _sc_agg(y_hbm, src_hbm, dst_hbm, zrows_hbm, zdeg_hbm,
            acc_out, deg_out,
            dst_v, src0_v, src1_v, rows0_v, rows1_v, ones_v, acc_sh, deg_sh,
            g0sem, g1sem, sl0sem, sl1sem, dsem):
    c = lax.axis_index("c")
    s = lax.axis_index("s")
    w = c * _NS + s

    pltpu.sync_copy(dst_hbm.at[w], dst_v)
    for i in range(_C // 16):
        ones_v[pl.ds(i * 16, 16)] = jnp.ones((16,), jnp.float32)

    sl = pl.ds(pl.multiple_of(s * _RPT, _RPT), _RPT)
    pltpu.sync_copy(zrows_hbm, acc_sh.at[sl])
    pltpu.sync_copy(zdeg_hbm, deg_sh.at[sl])
    plsc.subcore_barrier()

    def _srcld(k, srcb, sem):
        return pltpu.make_async_copy(
            src_hbm.at[pl.ds(pl.multiple_of(w * _EPW + k * _C, _C), _C)],
            srcb, sem)

    def _gather(srcb, rows, sem):
        return pltpu.make_async_copy(y_hbm.at[srcb], rows, sem)

    def _half(k, rows_a, gsem_a, srcb_a, slsem_a,
              rows_b, gsem_b, srcb_b, slsem_b):
        _srcld(k + 1, srcb_b, slsem_b).wait()
        _gather(srcb_a, rows_a, gsem_a).wait()
        _gather(srcb_b, rows_b, gsem_b).start()
        _srcld(k + 2, srcb_a, slsem_a).start()
        pltpu.sync_copy(rows_a, acc_sh.at[dst_v.at[k]], add=True)
        pltpu.sync_copy(ones_v, deg_sh.at[dst_v.at[k]], add=True)

    _srcld(0, src0_v, sl0sem).start()
    _srcld(0, src0_v, sl0sem).wait()
    _gather(src0_v, rows0_v, g0sem).start()
    _srcld(1, src1_v, sl1sem).start()

    def body(j, carry):
        _half(2 * j, rows0_v, g0sem, src0_v, sl0sem,
              rows1_v, g1sem, src1_v, sl1sem)
        _half(2 * j + 1, rows1_v, g1sem, src1_v, sl1sem,
              rows0_v, g0sem, src0_v, sl0sem)
        return carry

    lax.fori_loop(0, (_CHUNKS - 1) // 2, body, 0)
    kl = _CHUNKS - 1
    _gather(src0_v, rows0_v, g0sem).wait()
    _srcld(kl + 1, src1_v, sl1sem).wait()
    pltpu.sync_copy(rows0_v, acc_sh.at[dst_v.at[kl]], add=True)
    pltpu.sync_copy(ones_v, deg_sh.at[dst_v.at[kl]], add=True)

    plsc.subcore_barrier()
    osl = pl.ds(pl.multiple_of(c * _NPAD + s * _RPT, _RPT), _RPT)
    pltpu.sync_copy(acc_sh.at[sl], acc_out.at[osl])
    pltpu.sync_copy(deg_sh.at[sl], deg_out.at[osl])


def _post_body(xt_ref, acc_ref, deg_ref, norm_ref, td_ref, out_ref):
    inv = 1.0 / norm_ref[0, 0]
    agg = (acc_ref[0] + acc_ref[1])[:_N]
    deg = (deg_ref[0] + deg_ref[1])[:_N]
    scale = lax.rsqrt(deg + 1.0)
    out_ref[...] = xt_ref[...] * inv + td_ref[0, 0] * jnp.tanh(agg * scale)


_post = pl.pallas_call(
    _post_body,
    in_specs=[
        pl.BlockSpec(memory_space=pltpu.VMEM),
        pl.BlockSpec(memory_space=pltpu.VMEM),
        pl.BlockSpec(memory_space=pltpu.VMEM),
        pl.BlockSpec(memory_space=pltpu.SMEM),
        pl.BlockSpec(memory_space=pltpu.SMEM),
    ],
    out_shape=jax.ShapeDtypeStruct((_N, _D), jnp.float32),
)


def kernel(adj_his, t_diff, xu_t_plus, xi_t_plus, xu_embed, xi_embed, W):
    xt = jnp.concatenate([xu_t_plus, xi_t_plus], axis=0)
    xe = jnp.concatenate([xu_embed, xi_embed], axis=0)
    adj = adj_his.astype(jnp.int32)
    src = jnp.concatenate([adj[0], jnp.zeros((_C,), jnp.int32)])
    dst = adj[1].reshape(_NW, _CHUNKS, _C)

    y, norm = _pre(xt, xe, W)
    zrows = jnp.zeros((_RPT, _D), jnp.float32)
    zdeg = jnp.zeros((_RPT,), jnp.float32)
    acc, deg = _sc_agg(y, src, dst, zrows, zdeg)
    acc = acc.reshape(_NC, _NPAD, _D)

    out = _post(xt, acc, deg.reshape(_NC, _NPAD, 1),
                norm, t_diff.reshape(1, 1))
    return out[:_N_USER], out[_N_USER:]


# speedup vs baseline: 10.1550x; 1.0815x over previous
import functools

import jax
import jax.numpy as jnp
from jax import lax
from jax.experimental import pallas as pl
from jax.experimental.pallas import tpu as pltpu, tpu_sc as plsc

_N_USER = 6000
_N_ITEM = 4000
_N = _N_USER + _N_ITEM
_D = 128
_E = 320000

_NC = 2
_NS = 16
_NW = _NC * _NS
_EPW = _E // _NW
_C = 80
_CHUNKS = _EPW // _C
_NPAD = 10240
_RPT = _NPAD // _NS


def _pre_body(xut_ref, xit_ref, xue_ref, xie_ref, w_ref, y_ref, norm_ref):
    xu = xut_ref[...]
    xi = xit_ref[...]
    nsq = jnp.maximum(jnp.max(jnp.sum(xu * xu, axis=1)),
                      jnp.max(jnp.sum(xi * xi, axis=1)))
    norm = jnp.sqrt(nsq)
    inv = 1.0 / norm
    w = w_ref[...]
    y_ref[:_N_USER] = jnp.dot(xue_ref[...], w,
                              preferred_element_type=jnp.float32) * inv
    y_ref[_N_USER:] = jnp.dot(xie_ref[...], w,
                              preferred_element_type=jnp.float32) * inv
    norm_ref[...] = jnp.full((1, 1), norm, jnp.float32)


_pre = pl.pallas_call(
    _pre_body,
    out_shape=(
        jax.ShapeDtypeStruct((_N, _D), jnp.float32),
        jax.ShapeDtypeStruct((1, 1), jnp.float32),
    ),
)


_mesh = plsc.VectorSubcoreMesh(
    core_axis_name="c", subcore_axis_name="s", num_cores=_NC, num_subcores=_NS
)


@functools.partial(
    pl.kernel,
    out_type=(
        jax.ShapeDtypeStruct((_NC * _NPAD, _D), jnp.float32),
        jax.ShapeDtypeStruct((_NC * _NPAD,), jnp.float32),
    ),
    mesh=_mesh,
    scratch_types=[
        pltpu.VMEM((_CHUNKS, _C), jnp.int32),
        pltpu.VMEM((_C,), jnp.int32),
        pltpu.VMEM((_C,), jnp.int32),
        pltpu.VMEM((_C, _D), jnp.float32),
        pltpu.VMEM((_C, _D), jnp.float32),
        pltpu.VMEM((_C,), jnp.float32),
        pltpu.VMEM_SHARED((_NPAD, _D), jnp.float32),
        pltpu.VMEM_SHARED((_NPAD,), jnp.float32),
        pltpu.SemaphoreType.DMA,
        pltpu.SemaphoreType.DMA,
        pltpu.SemaphoreType.DMA,
        pltpu.SemaphoreType.DMA,
        pltpu.SemaphoreType.DMA,
    ],
)
def _sc_agg(y_hbm, src_hbm, dst_hbm, zrows_hbm, zdeg_hbm,
            acc_out, deg_out,
            dst_v, src0_v, src1_v, rows0_v, rows1_v, ones_v, acc_sh, deg_sh,
            g0sem, g1sem, sl0sem, sl1sem, dsem):
    c = lax.axis_index("c")
    s = lax.axis_index("s")
    w = c * _NS + s

    pltpu.sync_copy(dst_hbm.at[w], dst_v)
    for i in range(_C // 16):
        ones_v[pl.ds(i * 16, 16)] = jnp.ones((16,), jnp.float32)

    sl = pl.ds(pl.multiple_of(s * _RPT, _RPT), _RPT)
    pltpu.sync_copy(zrows_hbm, acc_sh.at[sl])
    pltpu.sync_copy(zdeg_hbm, deg_sh.at[sl])
    plsc.subcore_barrier()

    def _srcld(k, srcb, sem):
        return pltpu.make_async_copy(
            src_hbm.at[pl.ds(pl.multiple_of(w * _EPW + k * _C, _C), _C)],
            srcb, sem)

    def _gather(srcb, rows, sem):
        return pltpu.make_async_copy(y_hbm.at[srcb], rows, sem)

    def _half(k, rows_a, gsem_a, srcb_a, slsem_a,
              rows_b, gsem_b, srcb_b, slsem_b):
        _srcld(k + 1, srcb_b, slsem_b).wait()
        _gather(srcb_a, rows_a, gsem_a).wait()
        _gather(srcb_b, rows_b, gsem_b).start()
        @pl.when(k + 2 < _CHUNKS)
        def _():
            _srcld(k + 2, srcb_a, slsem_a).start()
        pltpu.sync_copy(rows_a, acc_sh.at[dst_v.at[k]], add=True)
        pltpu.sync_copy(ones_v, deg_sh.at[dst_v.at[k]], add=True)

    _srcld(0, src0_v, sl0sem).start()
    _srcld(0, src0_v, sl0sem).wait()
    _gather(src0_v, rows0_v, g0sem).start()
    _srcld(1, src1_v, sl1sem).start()

    def body(j, carry):
        _half(2 * j, rows0_v, g0sem, src0_v, sl0sem,
              rows1_v, g1sem, src1_v, sl1sem)
        _half(2 * j + 1, rows1_v, g1sem, src1_v, sl1sem,
              rows0_v, g0sem, src0_v, sl0sem)
        return carry

    lax.fori_loop(0, (_CHUNKS - 1) // 2, body, 0)
    kl = _CHUNKS - 1
    _gather(src0_v, rows0_v, g0sem).wait()
    pltpu.sync_copy(rows0_v, acc_sh.at[dst_v.at[kl]], add=True)
    pltpu.sync_copy(ones_v, deg_sh.at[dst_v.at[kl]], add=True)

    plsc.subcore_barrier()
    osl = pl.ds(pl.multiple_of(c * _NPAD + s * _RPT, _RPT), _RPT)
    pltpu.sync_copy(acc_sh.at[sl], acc_out.at[osl])
    pltpu.sync_copy(deg_sh.at[sl], deg_out.at[osl])


def _post_body(xut_ref, xit_ref, acc_ref, deg_ref, norm_ref, td_ref,
               xu_out_ref, xi_out_ref):
    inv = 1.0 / norm_ref[0, 0]
    td = td_ref[0, 0]
    agg = (acc_ref[0] + acc_ref[1])[:_N]
    deg = (deg_ref[0] + deg_ref[1])[:_N]
    scale = lax.rsqrt(deg + 1.0)
    upd = jnp.tanh(agg * scale)
    xu_out_ref[...] = xut_ref[...] * inv + td * upd[:_N_USER]
    xi_out_ref[...] = xit_ref[...] * inv + td * upd[_N_USER:]


_post = pl.pallas_call(
    _post_body,
    in_specs=[
        pl.BlockSpec(memory_space=pltpu.VMEM),
        pl.BlockSpec(memory_space=pltpu.VMEM),
        pl.BlockSpec(memory_space=pltpu.VMEM),
        pl.BlockSpec(memory_space=pltpu.VMEM),
        pl.BlockSpec(memory_space=pltpu.SMEM),
        pl.BlockSpec(memory_space=pltpu.SMEM),
    ],
    out_shape=(
        jax.ShapeDtypeStruct((_N_USER, _D), jnp.float32),
        jax.ShapeDtypeStruct((_N_ITEM, _D), jnp.float32),
    ),
)


def kernel(adj_his, t_diff, xu_t_plus, xi_t_plus, xu_embed, xi_embed, W):
    adj = adj_his.astype(jnp.int32)
    src = adj[0]
    dst = adj[1].reshape(_NW, _CHUNKS, _C)

    y, norm = _pre(xu_t_plus, xi_t_plus, xu_embed, xi_embed, W)
    zrows = jnp.zeros((_RPT, _D), jnp.float32)
    zdeg = jnp.zeros((_RPT,), jnp.float32)
    acc, deg = _sc_agg(y, src, dst, zrows, zdeg)

    return _post(xu_t_plus, xi_t_plus, acc.reshape(_NC, _NPAD, _D),
                 deg.reshape(_NC, _NPAD, 1), norm, t_diff.reshape(1, 1))


# speedup vs baseline: 10.3032x; 1.0146x over previous
import functools

import jax
import jax.numpy as jnp
from jax import lax
from jax.experimental import pallas as pl
from jax.experimental.pallas import tpu as pltpu, tpu_sc as plsc

_N_USER = 6000
_N_ITEM = 4000
_N = _N_USER + _N_ITEM
_D = 128
_E = 320000

_NC = 2
_NS = 16
_NW = _NC * _NS
_EPW = _E // _NW
_C = 80
_CHUNKS = _EPW // _C
_NPAD = 10240
_RPT = _NPAD // _NS


_mesh = plsc.VectorSubcoreMesh(
    core_axis_name="c", subcore_axis_name="s", num_cores=_NC, num_subcores=_NS
)


@functools.partial(
    pl.kernel,
    out_type=(
        jax.ShapeDtypeStruct((_NC * _NPAD, _D), jnp.float32),
        jax.ShapeDtypeStruct((_NC * _NPAD,), jnp.float32),
    ),
    mesh=_mesh,
    scratch_types=[
        pltpu.VMEM((_CHUNKS, _C), jnp.int32),
        pltpu.VMEM((_C,), jnp.int32),
        pltpu.VMEM((_C,), jnp.int32),
        pltpu.VMEM((_C, _D), jnp.float32),
        pltpu.VMEM((_C, _D), jnp.float32),
        pltpu.VMEM((_C,), jnp.float32),
        pltpu.VMEM_SHARED((_NPAD, _D), jnp.float32),
        pltpu.VMEM_SHARED((_NPAD,), jnp.float32),
        pltpu.SemaphoreType.DMA,
        pltpu.SemaphoreType.DMA,
        pltpu.SemaphoreType.DMA,
        pltpu.SemaphoreType.DMA,
        pltpu.SemaphoreType.DMA,
    ],
)
def _sc_agg(y_hbm, src_hbm, dst_hbm, zrows_hbm, zdeg_hbm,
            acc_out, deg_out,
            dst_v, src0_v, src1_v, rows0_v, rows1_v, ones_v, acc_sh, deg_sh,
            g0sem, g1sem, sl0sem, sl1sem, dsem):
    c = lax.axis_index("c")
    s = lax.axis_index("s")
    w = c * _NS + s

    pltpu.sync_copy(dst_hbm.at[w], dst_v)
    for i in range(_C // 16):
        ones_v[pl.ds(i * 16, 16)] = jnp.ones((16,), jnp.float32)

    sl = pl.ds(pl.multiple_of(s * _RPT, _RPT), _RPT)
    pltpu.sync_copy(zrows_hbm, acc_sh.at[sl])
    pltpu.sync_copy(zdeg_hbm, deg_sh.at[sl])
    plsc.subcore_barrier()

    def _srcld(k, srcb, sem):
        return pltpu.make_async_copy(
            src_hbm.at[pl.ds(pl.multiple_of(w * _EPW + k * _C, _C), _C)],
            srcb, sem)

    def _gather(srcb, rows, sem):
        return pltpu.make_async_copy(y_hbm.at[srcb], rows, sem)

    def _half(k, rows_a, gsem_a, srcb_a, slsem_a,
              rows_b, gsem_b, srcb_b, slsem_b):
        _srcld(k + 1, srcb_b, slsem_b).wait()
        _gather(srcb_a, rows_a, gsem_a).wait()
        _gather(srcb_b, rows_b, gsem_b).start()
        @pl.when(k + 2 < _CHUNKS)
        def _():
            _srcld(k + 2, srcb_a, slsem_a).start()
        pltpu.sync_copy(rows_a, acc_sh.at[dst_v.at[k]], add=True)
        pltpu.sync_copy(ones_v, deg_sh.at[dst_v.at[k]], add=True)

    _srcld(0, src0_v, sl0sem).start()
    _srcld(0, src0_v, sl0sem).wait()
    _gather(src0_v, rows0_v, g0sem).start()
    _srcld(1, src1_v, sl1sem).start()

    def body(j, carry):
        _half(2 * j, rows0_v, g0sem, src0_v, sl0sem,
              rows1_v, g1sem, src1_v, sl1sem)
        _half(2 * j + 1, rows1_v, g1sem, src1_v, sl1sem,
              rows0_v, g0sem, src0_v, sl0sem)
        return carry

    lax.fori_loop(0, (_CHUNKS - 1) // 2, body, 0)
    kl = _CHUNKS - 1
    _gather(src0_v, rows0_v, g0sem).wait()
    pltpu.sync_copy(rows0_v, acc_sh.at[dst_v.at[kl]], add=True)
    pltpu.sync_copy(ones_v, deg_sh.at[dst_v.at[kl]], add=True)

    plsc.subcore_barrier()
    osl = pl.ds(pl.multiple_of(c * _NPAD + s * _RPT, _RPT), _RPT)
    pltpu.sync_copy(acc_sh.at[sl], acc_out.at[osl])
    pltpu.sync_copy(deg_sh.at[sl], deg_out.at[osl])


def _post_body(xut_ref, xit_ref, acc_ref, deg_ref, w_ref, td_ref,
               xu_out_ref, xi_out_ref):
    xu = xut_ref[...]
    xi = xit_ref[...]
    nsq = jnp.maximum(jnp.max(jnp.sum(xu * xu, axis=1)),
                      jnp.max(jnp.sum(xi * xi, axis=1)))
    inv = lax.rsqrt(nsq)
    td = td_ref[0, 0]
    agg = (acc_ref[0] + acc_ref[1])[:_N]
    deg = (deg_ref[0] + deg_ref[1])[:_N]
    scale = lax.rsqrt(deg + 1.0) * inv
    md = jnp.dot(agg, w_ref[...], preferred_element_type=jnp.float32)
    upd = jnp.tanh(md * scale)
    xu_out_ref[...] = xu * inv + td * upd[:_N_USER]
    xi_out_ref[...] = xi * inv + td * upd[_N_USER:]


_post = pl.pallas_call(
    _post_body,
    in_specs=[
        pl.BlockSpec(memory_space=pltpu.VMEM),
        pl.BlockSpec(memory_space=pltpu.VMEM),
        pl.BlockSpec(memory_space=pltpu.VMEM),
        pl.BlockSpec(memory_space=pltpu.VMEM),
        pl.BlockSpec(memory_space=pltpu.VMEM),
        pl.BlockSpec(memory_space=pltpu.SMEM),
    ],
    out_shape=(
        jax.ShapeDtypeStruct((_N_USER, _D), jnp.float32),
        jax.ShapeDtypeStruct((_N_ITEM, _D), jnp.float32),
    ),
)


def kernel(adj_his, t_diff, xu_t_plus, xi_t_plus, xu_embed, xi_embed, W):
    adj = adj_his.astype(jnp.int32)
    src = adj[0]
    dst = adj[1].reshape(_NW, _CHUNKS, _C)
    xe = jnp.concatenate([xu_embed, xi_embed], axis=0)

    zrows = jnp.zeros((_RPT, _D), jnp.float32)
    zdeg = jnp.zeros((_RPT,), jnp.float32)
    acc, deg = _sc_agg(xe, src, dst, zrows, zdeg)

    return _post(xu_t_plus, xi_t_plus, acc.reshape(_NC, _NPAD, _D),
                 deg.reshape(_NC, _NPAD, 1), W, t_diff.reshape(1, 1))


# speedup vs baseline: 13.8191x; 1.3412x over previous
import functools

import jax
import jax.numpy as jnp
from jax import lax
from jax.experimental import pallas as pl
from jax.experimental.pallas import tpu as pltpu, tpu_sc as plsc

_N_USER = 6000
_N_ITEM = 4000
_N = _N_USER + _N_ITEM
_D = 128
_E = 320000

_NC = 2
_NS = 16
_NW = _NC * _NS
_EPW = _E // _NW
_C = 80
_CHUNKS = _EPW // _C
_NPAD = 10240
_RPT = _NPAD // _NS


_mesh = plsc.VectorSubcoreMesh(
    core_axis_name="c", subcore_axis_name="s", num_cores=_NC, num_subcores=_NS
)


@functools.partial(
    pl.kernel,
    out_type=(
        jax.ShapeDtypeStruct((_NC * _NPAD, _D), jnp.float32),
        jax.ShapeDtypeStruct((_NC * _NPAD,), jnp.float32),
    ),
    mesh=_mesh,
    scratch_types=[
        pltpu.VMEM((_CHUNKS, _C), jnp.int32),
        [pltpu.VMEM((_C,), jnp.int32) for _ in range(3)],
        [pltpu.VMEM((_C, _D), jnp.float32) for _ in range(3)],
        pltpu.VMEM((_C,), jnp.float32),
        pltpu.VMEM_SHARED((_NPAD, _D), jnp.float32),
        pltpu.VMEM_SHARED((_NPAD,), jnp.float32),
        [pltpu.SemaphoreType.DMA for _ in range(3)],
        [pltpu.SemaphoreType.DMA for _ in range(3)],
    ],
)
def _sc_agg(y_hbm, src_hbm, dst_hbm, zrows_hbm, zdeg_hbm,
            acc_out, deg_out,
            dst_v, srcb, rowsb, ones_v, acc_sh, deg_sh, gsem, slsem):
    c = lax.axis_index("c")
    s = lax.axis_index("s")
    w = c * _NS + s

    pltpu.sync_copy(dst_hbm.at[w], dst_v)
    for i in range(_C // 16):
        ones_v[pl.ds(i * 16, 16)] = jnp.ones((16,), jnp.float32)

    sl = pl.ds(pl.multiple_of(s * _RPT, _RPT), _RPT)
    pltpu.sync_copy(zrows_hbm, acc_sh.at[sl])
    pltpu.sync_copy(zdeg_hbm, deg_sh.at[sl])
    plsc.subcore_barrier()

    def _srcld(k, srcb, sem):
        return pltpu.make_async_copy(
            src_hbm.at[pl.ds(pl.multiple_of(w * _EPW + k * _C, _C), _C)],
            srcb, sem)

    def _gather(srcb, rows, sem):
        return pltpu.make_async_copy(y_hbm.at[srcb], rows, sem)

    def _scatter(k, a):
        pltpu.sync_copy(rowsb[a], acc_sh.at[dst_v.at[k]], add=True)
        pltpu.sync_copy(ones_v, deg_sh.at[dst_v.at[k]], add=True)

    def _third(k, a, b, c2, last):
        _srcld(k + 2, srcb[c2], slsem[c2]).wait()
        _gather(srcb[a], rowsb[a], gsem[a]).wait()
        _gather(srcb[c2], rowsb[c2], gsem[c2]).start()
        if last:
            @pl.when(k + 3 < _CHUNKS)
            def _():
                _srcld(k + 3, srcb[a], slsem[a]).start()
        else:
            _srcld(k + 3, srcb[a], slsem[a]).start()
        _scatter(k, a)

    _srcld(0, srcb[0], slsem[0]).start()
    _srcld(1, srcb[1], slsem[1]).start()
    _srcld(2, srcb[2], slsem[2]).start()
    _srcld(0, srcb[0], slsem[0]).wait()
    _gather(srcb[0], rowsb[0], gsem[0]).start()
    _srcld(1, srcb[1], slsem[1]).wait()
    _gather(srcb[1], rowsb[1], gsem[1]).start()

    def body(j, carry):
        _third(3 * j, 0, 1, 2, False)
        _third(3 * j + 1, 1, 2, 0, False)
        _third(3 * j + 2, 2, 0, 1, True)
        return carry

    lax.fori_loop(0, (_CHUNKS - 2) // 3, body, 0)
    ka = _CHUNKS - 2
    kb = _CHUNKS - 1
    _gather(srcb[0], rowsb[0], gsem[0]).wait()
    _scatter(ka, 0)
    _gather(srcb[1], rowsb[1], gsem[1]).wait()
    _scatter(kb, 1)

    plsc.subcore_barrier()
    osl = pl.ds(pl.multiple_of(c * _NPAD + s * _RPT, _RPT), _RPT)
    pltpu.sync_copy(acc_sh.at[sl], acc_out.at[osl])
    pltpu.sync_copy(deg_sh.at[sl], deg_out.at[osl])


def _post_body(xut_ref, xit_ref, acc_ref, deg_ref, w_ref, td_ref,
               xu_out_ref, xi_out_ref):
    xu = xut_ref[...]
    xi = xit_ref[...]
    nsq = jnp.maximum(jnp.max(jnp.sum(xu * xu, axis=1)),
                      jnp.max(jnp.sum(xi * xi, axis=1)))
    inv = lax.rsqrt(nsq)
    td = td_ref[0, 0]
    agg = (acc_ref[0] + acc_ref[1])[:_N]
    deg = (deg_ref[0] + deg_ref[1])[:_N]
    scale = lax.rsqrt(deg + 1.0) * inv
    md = jnp.dot(agg, w_ref[...], preferred_element_type=jnp.float32)
    upd = jnp.tanh(md * scale)
    xu_out_ref[...] = xu * inv + td * upd[:_N_USER]
    xi_out_ref[...] = xi * inv + td * upd[_N_USER:]


_post = pl.pallas_call(
    _post_body,
    in_specs=[
        pl.BlockSpec(memory_space=pltpu.VMEM),
        pl.BlockSpec(memory_space=pltpu.VMEM),
        pl.BlockSpec(memory_space=pltpu.VMEM),
        pl.BlockSpec(memory_space=pltpu.VMEM),
        pl.BlockSpec(memory_space=pltpu.VMEM),
        pl.BlockSpec(memory_space=pltpu.SMEM),
    ],
    out_shape=(
        jax.ShapeDtypeStruct((_N_USER, _D), jnp.float32),
        jax.ShapeDtypeStruct((_N_ITEM, _D), jnp.float32),
    ),
)


def kernel(adj_his, t_diff, xu_t_plus, xi_t_plus, xu_embed, xi_embed, W):
    adj = adj_his.astype(jnp.int32)
    src = adj[0]
    dst = adj[1].reshape(_NW, _CHUNKS, _C)
    xe = jnp.concatenate([xu_embed, xi_embed], axis=0)

    zrows = jnp.zeros((_RPT, _D), jnp.float32)
    zdeg = jnp.zeros((_RPT,), jnp.float32)
    acc, deg = _sc_agg(xe, src, dst, zrows, zdeg)

    return _post(xu_t_plus, xi_t_plus, acc.reshape(_NC, _NPAD, _D),
                 deg.reshape(_NC, _NPAD, 1), W, t_diff.reshape(1, 1))
